# baseline (device time: 18027 ns/iter reference)
import jax
import jax.numpy as jnp
from jax import lax
from jax.experimental import pallas as pl
from jax.experimental.pallas import tpu as pltpu

N_DEV = 8
CH = 64


def kernel(x):
    m, n = x.shape

    def body(x_ref, out_ref, conv_ref, rs_buf,
             rs_send, rs_recv, ag_send, ag_recv):
        my = lax.axis_index("i")

        barrier_sem = pltpu.get_barrier_semaphore()
        for e in range(1, N_DEV):
            pl.semaphore_signal(
                barrier_sem, inc=1,
                device_id=(my ^ e,), device_id_type=pl.DeviceIdType.MESH,
            )
        conv_ref[:] = x_ref[:].astype(jnp.bfloat16)
        pl.semaphore_wait(barrier_sem, N_DEV - 1)

        rs = []
        for e in range(1, N_DEV):
            peer = my ^ e
            rdma = pltpu.make_async_remote_copy(
                src_ref=conv_ref.at[pl.ds(peer * CH, CH), :],
                dst_ref=rs_buf.at[e - 1],
                send_sem=rs_send.at[e - 1],
                recv_sem=rs_recv.at[e - 1],
                device_id=(peer,),
                device_id_type=pl.DeviceIdType.MESH,
            )
            rdma.start()
            rs.append(rdma)

        acc = conv_ref[pl.ds(my * CH, CH), :]
        for e in range(1, N_DEV):
            rs[e - 1].wait_recv()
            acc = acc + rs_buf[e - 1]
        out_ref[pl.ds(my * CH, CH), :] = acc

        ag = []
        for e in range(1, N_DEV):
            peer = my ^ e
            rdma = pltpu.make_async_remote_copy(
                src_ref=out_ref.at[pl.ds(my * CH, CH), :],
                dst_ref=out_ref.at[pl.ds(my * CH, CH), :],
                send_sem=ag_send.at[e - 1],
                recv_sem=ag_recv.at[e - 1],
                device_id=(peer,),
                device_id_type=pl.DeviceIdType.MESH,
            )
            rdma.start()
            ag.append(rdma)

        for e in range(1, N_DEV):
            ag[e - 1].wait_recv()
        for e in range(1, N_DEV):
            rs[e - 1].wait_send()
            ag[e - 1].wait_send()

    return pl.pallas_call(
        body,
        out_shape=jax.ShapeDtypeStruct((m, n), jnp.bfloat16),
        in_specs=[pl.BlockSpec(memory_space=pltpu.VMEM)],
        out_specs=pl.BlockSpec(memory_space=pltpu.VMEM),
        scratch_shapes=[
            pltpu.VMEM((m, n), jnp.bfloat16),
            pltpu.VMEM((N_DEV - 1, CH, n), jnp.bfloat16),
            pltpu.SemaphoreType.DMA((N_DEV - 1,)),
            pltpu.SemaphoreType.DMA((N_DEV - 1,)),
            pltpu.SemaphoreType.DMA((N_DEV - 1,)),
            pltpu.SemaphoreType.DMA((N_DEV - 1,)),
        ],
        compiler_params=pltpu.CompilerParams(collective_id=0),
    )(x)


# device time: 16386 ns/iter; 1.1001x vs baseline; 1.1001x over previous
import jax
import jax.numpy as jnp
from jax import lax
from jax.experimental import pallas as pl
from jax.experimental.pallas import tpu as pltpu

N_DEV = 8
SLOTS = 3
MASKS = (1, 3, 4)
ROW_PARTS = ((0, 176), (176, 176), (352, 160))
ORDERS = ((1, 3, 4), (3, 4, 1), (4, 1, 3))


def kernel(x):
    m, n = x.shape

    def body(x_ref, out_ref, recv_buf, send_sems, recv_sems):
        my = lax.axis_index("i")

        barrier_sem = pltpu.get_barrier_semaphore()
        for mask in MASKS:
            pl.semaphore_signal(
                barrier_sem, inc=1,
                device_id=(my ^ mask,), device_id_type=pl.DeviceIdType.MESH,
            )
        for start, size in ROW_PARTS:
            rows = pl.ds(start, size)
            out_ref[rows, :] = x_ref[rows, :].astype(jnp.bfloat16)
        pl.semaphore_wait(barrier_sem, len(MASKS))

        def start_part(p, s):
            start, size = ROW_PARTS[p]
            rdma = pltpu.make_async_remote_copy(
                src_ref=out_ref.at[pl.ds(start, size), :],
                dst_ref=recv_buf.at[s, pl.ds(start, size), :],
                send_sem=send_sems.at[s, p],
                recv_sem=recv_sems.at[s, p],
                device_id=(my ^ ORDERS[p][s],),
                device_id_type=pl.DeviceIdType.MESH,
            )
            rdma.start()
            return rdma

        rdmas = [start_part(p, 0) for p in range(len(ROW_PARTS))]
        for s in range(SLOTS):
            for p, (start, size) in enumerate(ROW_PARTS):
                rdmas[p].wait()
                rows = pl.ds(start, size)
                out_ref[rows, :] = out_ref[rows, :] + recv_buf[s, rows, :]
                if s + 1 < SLOTS:
                    rdmas[p] = start_part(p, s + 1)

    return pl.pallas_call(
        body,
        out_shape=jax.ShapeDtypeStruct((m, n), jnp.bfloat16),
        in_specs=[pl.BlockSpec(memory_space=pltpu.VMEM)],
        out_specs=pl.BlockSpec(memory_space=pltpu.VMEM),
        scratch_shapes=[
            pltpu.VMEM((SLOTS, m, n), jnp.bfloat16),
            pltpu.SemaphoreType.DMA((SLOTS, len(ROW_PARTS))),
            pltpu.SemaphoreType.DMA((SLOTS, len(ROW_PARTS))),
        ],
        compiler_params=pltpu.CompilerParams(collective_id=0),
    )(x)
